# baseline (speedup 1.0000x reference)
import jax
import jax.numpy as jnp
from jax.experimental import pallas as pl


def _vals_kernel(t_eval_ref, t_ref, dt_ref, y_ref, y_next_ref, idx_ref,
                 vals_ref):
    b_blk, T = t_eval_ref.shape
    te_tab = t_eval_ref[...]
    idx = idx_ref[...]
    cols = jax.lax.broadcasted_iota(jnp.int32, (b_blk, T), 1)
    te = jnp.sum(jnp.where(cols == idx, te_tab, 0.0), axis=1, keepdims=True)
    theta = jnp.clip((te - t_ref[...]) / dt_ref[...], 0.0, 1.0)
    vals_ref[...] = y_ref[...] * (1.0 - theta) + y_next_ref[...] * theta


def _pair_kernel(vals_ref, idxe_ref, idxo_ref, out_ref):
    T = out_ref.shape[0]
    P = vals_ref.shape[0]
    v = vals_ref[...]
    ie = idxe_ref[...]
    io = idxo_ref[...]
    li = jax.lax.broadcasted_iota(jnp.int32, (P, 128), 1)
    left = li < 64
    for ti in range(T):
        fe = (ie == ti).astype(jnp.float32)
        fo = (io == ti).astype(jnp.float32)
        fe128 = jnp.broadcast_to(fe, (P, 128))
        fo128 = jnp.broadcast_to(fo, (P, 128))
        mult = jnp.where(left, fe128, fo128)
        out_ref[ti, :, :] = v * mult


def kernel(y_eval, t_eval, t, dt, y, y_next, eval_t_idx, sample_idx):
    T, B, D = y_eval.shape
    B_BLK = 1024
    vals = pl.pallas_call(
        _vals_kernel,
        grid=(B // B_BLK,),
        in_specs=[
            pl.BlockSpec((B_BLK, T), lambda b: (b, 0)),
            pl.BlockSpec((B_BLK, 1), lambda b: (b, 0)),
            pl.BlockSpec((B_BLK, 1), lambda b: (b, 0)),
            pl.BlockSpec((B_BLK, D), lambda b: (b, 0)),
            pl.BlockSpec((B_BLK, D), lambda b: (b, 0)),
            pl.BlockSpec((B_BLK, 1), lambda b: (b, 0)),
        ],
        out_specs=pl.BlockSpec((B_BLK, D), lambda b: (b, 0)),
        out_shape=jax.ShapeDtypeStruct((B, D), jnp.float32),
    )(t_eval, t[:, None], dt[:, None], y, y_next, eval_t_idx[:, None])

    vals128 = vals.reshape(B // 2, 2 * D)
    idx2 = eval_t_idx.reshape(B // 2, 2)
    idxe = idx2[:, 0:1]
    idxo = idx2[:, 1:2]

    P_BLK = 512
    out = pl.pallas_call(
        _pair_kernel,
        grid=(B // 2 // P_BLK,),
        in_specs=[
            pl.BlockSpec((P_BLK, 2 * D), lambda b: (b, 0)),
            pl.BlockSpec((P_BLK, 1), lambda b: (b, 0)),
            pl.BlockSpec((P_BLK, 1), lambda b: (b, 0)),
        ],
        out_specs=pl.BlockSpec((T, P_BLK, 2 * D), lambda b: (0, b, 0)),
        out_shape=jax.ShapeDtypeStruct((T, B // 2, 2 * D), jnp.float32),
    )(vals128, idxe, idxo)
    return out.reshape(T, B, D)


# speedup vs baseline: 1.7728x; 1.7728x over previous
import functools

import jax
import jax.numpy as jnp
from jax import lax
from jax.experimental import pallas as pl
from jax.experimental.pallas import tpu as pltpu
from jax.experimental.pallas import tpu_sc as plsc

NC, NS, L = 2, 16, 16
NW = NC * NS


def _sc_body(T, B, D, CHUNK,
             tef_hbm, t_hbm, dt_hbm, y_hbm, yn_hbm, idx_hbm, out_hbm,
             idx_v, t_v, dt_v, th_v, tef_v, yn_v, buf_v,
             zsem, ssem):
    wid = lax.axis_index("s") * NC + lax.axis_index("c")
    base = wid * CHUNK

    pltpu.sync_copy(idx_hbm.at[pl.ds(base, CHUNK)], idx_v)
    pltpu.sync_copy(t_hbm.at[pl.ds(base, CHUNK)], t_v)
    pltpu.sync_copy(dt_hbm.at[pl.ds(base, CHUNK)], dt_v)
    pltpu.sync_copy(tef_hbm.at[pl.ds(base * T, CHUNK * T)], tef_v)
    pltpu.sync_copy(yn_hbm.at[pl.ds(base * D, CHUNK * D)], yn_v)

    zrow = jnp.zeros((L,), jnp.float32)

    def zbody(i, _):
        for k in range(D // L):
            buf_v[i, pl.ds(k * L, L)] = zrow
        return 0
    lax.fori_loop(0, CHUNK, zbody, 0)

    zcopies = [
        pltpu.make_async_copy(
            buf_v, out_hbm.at[pl.ds(t * B + base, CHUNK)], zsem)
        for t in range(T)
    ]
    for c in zcopies:
        c.start()

    iota = lax.broadcasted_iota(jnp.int32, (L,), 0)
    for j in range(CHUNK // L):
        sl = pl.ds(j * L, L)
        fi16 = (iota + j * L) * T + idx_v[sl]
        te16 = plsc.load_gather(tef_v, [fi16])
        th = (te16 - t_v[sl]) / dt_v[sl]
        th_v[sl] = jnp.minimum(jnp.maximum(th, 0.0), 1.0)

    for c in zcopies:
        c.wait()

    pltpu.sync_copy(y_hbm.at[pl.ds(base, CHUNK)], buf_v)

    def fbody(i, _):
        th16 = plsc.load_gather(th_v, [jnp.zeros((L,), jnp.int32) + i])
        for k in range(D // L):
            sl = pl.ds(k * L, L)
            yv = buf_v[i, sl]
            ynv = yn_v[pl.ds(i * D + k * L, L)]
            buf_v[i, sl] = yv + th16 * (ynv - yv)
        return 0
    lax.fori_loop(0, CHUNK, fbody, 0)

    def sbody(j, _):
        idx16 = idx_v[pl.ds(j * L, L)]
        for l in range(L):
            i = j * L + l
            r = idx16[l] * B + base + i
            pltpu.make_async_copy(
                buf_v.at[pl.ds(i, 1)], out_hbm.at[pl.ds(r, 1)], ssem).start()
        return 0
    lax.fori_loop(0, CHUNK // L, sbody, 0)

    pltpu.make_async_copy(
        y_hbm.at[pl.ds(0, CHUNK)], buf_v, ssem).wait()


def kernel(y_eval, t_eval, t, dt, y, y_next, eval_t_idx, sample_idx):
    T, B, D = y_eval.shape
    CHUNK = B // NW
    mesh = plsc.VectorSubcoreMesh(
        core_axis_name="c", subcore_axis_name="s",
        num_cores=NC, num_subcores=NS)

    k = functools.partial(
        pl.kernel,
        out_type=jax.ShapeDtypeStruct((T * B, D), jnp.float32),
        mesh=mesh,
        scratch_types=[
            pltpu.VMEM((CHUNK,), jnp.int32),
            pltpu.VMEM((CHUNK,), jnp.float32),
            pltpu.VMEM((CHUNK,), jnp.float32),
            pltpu.VMEM((CHUNK,), jnp.float32),
            pltpu.VMEM((CHUNK * T,), jnp.float32),
            pltpu.VMEM((CHUNK * D,), jnp.float32),
            pltpu.VMEM((CHUNK, D), jnp.float32),
            pltpu.SemaphoreType.DMA,
            pltpu.SemaphoreType.DMA,
        ],
        compiler_params=pltpu.CompilerParams(needs_layout_passes=False),
    )(functools.partial(_sc_body, T, B, D, CHUNK))

    out = k(t_eval.reshape(B * T), t, dt, y,
            y_next.reshape(B * D), eval_t_idx)
    return out.reshape(T, B, D)
